# Initial kernel scaffold; baseline (speedup 1.0000x reference)
import jax
import jax.numpy as jnp
from jax.experimental import pallas as pl


def kernel(x_book, x_film, x_music, ei_book_film, ei_film_music, ei_music_book, W_in_book, b_in_book, W_k_book, b_k_book, W_q_book, b_q_book, W_v_book, b_v_book, W_a_book, b_a_book, skip_book, W_in_film, b_in_film, W_k_film, b_k_film, W_q_film, b_q_film, W_v_film, b_v_film, W_a_film, b_a_film, skip_film, W_in_music, b_in_music, W_k_music, b_k_music, W_q_music, b_q_music, W_v_music, b_v_music, W_a_music, b_a_music, skip_music, a_rel_book_film, m_rel_book_film, p_rel_book_film, a_rel_film_music, m_rel_film_music, p_rel_film_music, a_rel_music_book, m_rel_music_book, p_rel_music_book, W_out, b_out):
    raise NotImplementedError("write your pallas kernel here")


# speedup vs baseline: 51.7620x; 51.7620x over previous
import jax
import jax.numpy as jnp
from jax import lax
from jax.experimental import pallas as pl
from jax.experimental.pallas import tpu as pltpu
from jax.experimental.pallas import tpu_sc as plsc

N = 50000
E = 800000
C = 64
H = 8
D = 8
OUT = 8

B = 128
NSUB = 16
NW = 32
EPAD = 802816
CPW = EPAD // B // NW
CPT = EPAD // B // NSUB
ACCW = 72
NR = 25088
SENT = 2 ** 30
ZCH = NR // B
ZPT = -(-ZCH // NSUB)
RPT = NR // NSUB
EB = 2048


def _pre_body(xb_ref, xm_ref, Wib, bib, Wim, bim, Wq, bq, Wk, bk, Wv, bv,
              xbo, qto, kvo):
    xb = jnp.maximum(jnp.dot(xb_ref[...], Wib[...],
                             preferred_element_type=jnp.float32) + bib[...], 0.0)
    xm = jnp.maximum(jnp.dot(xm_ref[...], Wim[...],
                             preferred_element_type=jnp.float32) + bim[...], 0.0)
    xbo[...] = xb
    qto[:, :C] = jnp.dot(xb, Wq[...], preferred_element_type=jnp.float32) + bq[...]
    qto[:, C:] = jnp.zeros_like(xb)
    kvo[:, :C] = jnp.dot(xm, Wk[...], preferred_element_type=jnp.float32) + bk[...]
    kvo[:, C:] = jnp.dot(xm, Wv[...], preferred_element_type=jnp.float32) + bv[...]


def _tc_pre(x_book, x_music, Wib, bib, Wim, bim, Wq, bq, Wk, bk, Wv, bv,
            rb=1000):
    grid = (N // rb,)
    row = pl.BlockSpec((rb, C), lambda i: (i, 0))
    mat = pl.BlockSpec((C, C), lambda i: (0, 0))
    vec = pl.BlockSpec((1, C), lambda i: (0, 0))
    return pl.pallas_call(
        _pre_body,
        grid=grid,
        in_specs=[row, row, mat, vec, mat, vec, mat, vec, mat, vec, mat, vec],
        out_specs=[row, pl.BlockSpec((rb, 2 * C), lambda i: (i, 0)),
                   pl.BlockSpec((rb, 2 * C), lambda i: (i, 0))],
        out_shape=[jax.ShapeDtypeStruct((N, C), jnp.float32),
                   jax.ShapeDtypeStruct((N, 2 * C), jnp.float32),
                   jax.ShapeDtypeStruct((N, 2 * C), jnp.float32)],
    )(x_book, x_music, Wib, bib, Wim, bim, Wq, bq, Wk, bk, Wv, bv)


def _sc_gather_body(kvf, qf, srcg, dstg, kv_e, q_e,
                    sidx, didx, kvb, qb, sem0, sem1):
    c = lax.axis_index("c")
    s = lax.axis_index("s")
    w = s * 2 + c

    def body(i, carry):
        base = (w + i * NW) * B
        pltpu.sync_copy(srcg.at[pl.ds(base, B)], sidx)
        pltpu.sync_copy(dstg.at[pl.ds(base, B)], didx)
        cp0 = pltpu.async_copy(kvf.at[sidx], kvb, sem0)
        cp1 = pltpu.async_copy(qf.at[didx], qb, sem1)
        cp0.wait()
        cp1.wait()
        pltpu.sync_copy(kvb, kv_e.at[pl.ds(base, B)])
        pltpu.sync_copy(qb, q_e.at[pl.ds(base, B)])
        return carry

    lax.fori_loop(0, CPW, body, 0)


def _sc_gather(kvf, qf, srcg, dstg):
    mesh = plsc.VectorSubcoreMesh(core_axis_name="c", subcore_axis_name="s")
    f = pl.kernel(
        _sc_gather_body,
        out_type=[jax.ShapeDtypeStruct((EPAD, 2 * C), jnp.float32),
                  jax.ShapeDtypeStruct((EPAD, 2 * C), jnp.float32)],
        mesh=mesh,
        scratch_types=[
            pltpu.VMEM((B,), jnp.int32),
            pltpu.VMEM((B,), jnp.int32),
            pltpu.VMEM((B, 2 * C), jnp.float32),
            pltpu.VMEM((B, 2 * C), jnp.float32),
            pltpu.SemaphoreType.DMA,
            pltpu.SemaphoreType.DMA,
        ],
    )
    return f(kvf, qf, srcg, dstg)


def _mid_body(kv_ref, q_ref, summ, rep, o_ref):
    kv = kv_ref[...]
    q = q_ref[...]
    t = q[:, :C] * kv[:, :C]
    w8 = jnp.exp(jnp.dot(t, summ[...], preferred_element_type=jnp.float32))
    wv = kv[:, C:] * jnp.dot(w8, rep[...], preferred_element_type=jnp.float32)
    o_ref[:, :C] = wv
    o_ref[:, C:C + H] = w8
    o_ref[:, C + H:] = jnp.zeros((wv.shape[0], C - H), jnp.float32)


def _tc_mid(kv_e, q_e, summ, rep):
    grid = (EPAD // EB,)
    return pl.pallas_call(
        _mid_body,
        grid=grid,
        in_specs=[
            pl.BlockSpec((EB, 2 * C), lambda i: (i, 0)),
            pl.BlockSpec((EB, 2 * C), lambda i: (i, 0)),
            pl.BlockSpec((C, H), lambda i: (0, 0)),
            pl.BlockSpec((H, C), lambda i: (0, 0)),
        ],
        out_specs=pl.BlockSpec((EB, 2 * C), lambda i: (i, 0)),
        out_shape=jax.ShapeDtypeStruct((EPAD, 2 * C), jnp.float32),
    )(kv_e, q_e, summ, rep)


def _make_scatter_body(width, coff):
    zch = NR // B
    zpt = -(-zch // NSUB)
    rpt = NR // NSUB

    def body(rows, dsts, zrows, acc_out, dsc, ob, zb, acc, sem0):
        c = lax.axis_index("c")
        s = lax.axis_index("s")

        pltpu.sync_copy(zrows.at[:, pl.ds(0, width)], zb)

        def zbody(i, carry):
            j = jnp.minimum(s + i * NSUB, zch - 1)
            pltpu.sync_copy(zb, acc.at[pl.ds(j * B, B)])
            return carry

        lax.fori_loop(0, zpt, zbody, 0)
        plsc.subcore_barrier()

        def ebody(i, carry):
            base = (s + i * NSUB) * B
            pltpu.sync_copy(rows.at[pl.ds(base, B), pl.ds(coff, width)], ob)
            pltpu.sync_copy(dsts.at[pl.ds(c * EPAD + base, B)], dsc)
            pltpu.sync_copy(ob, acc.at[plsc.Indices(dsc, ignored_value=SENT)],
                            add=True)
            return carry

        lax.fori_loop(0, CPT, ebody, 0)
        plsc.subcore_barrier()
        pltpu.sync_copy(acc.at[pl.ds(s * rpt, rpt)],
                        acc_out.at[pl.ds(c * NR + s * rpt, rpt), pl.ds(0, width)])

    return body


def _sc_scatter(rows, dsts, zrows, width, coff):
    mesh = plsc.VectorSubcoreMesh(core_axis_name="c", subcore_axis_name="s")
    f = pl.kernel(
        _make_scatter_body(width, coff),
        out_type=jax.ShapeDtypeStruct((2 * NR, 2 * C), jnp.float32),
        mesh=mesh,
        compiler_params=pltpu.CompilerParams(use_tc_tiling_on_sc=False),
        scratch_types=[
            pltpu.VMEM((B,), jnp.int32),
            pltpu.VMEM((B, width), jnp.float32),
            pltpu.VMEM((B, width), jnp.float32),
            pltpu.VMEM_SHARED((NR, width), jnp.float32),
            pltpu.SemaphoreType.DMA,
        ],
    )
    return f(rows, dsts, zrows)


def _post_body(num_ref, den_ref, xb_ref, rep, Wa, ba, Woa, Wox, bo, yo):
    den_exp = jnp.dot(den_ref[...], rep[...], preferred_element_type=jnp.float32)
    agg = num_ref[...] / (den_exp + 1e-16)
    o = jax.nn.gelu(jnp.dot(agg, Wa[...], preferred_element_type=jnp.float32)
                    + ba[...])
    yo[...] = (jnp.dot(o, Woa[...], preferred_element_type=jnp.float32)
               + jnp.dot(xb_ref[...], Wox[...], preferred_element_type=jnp.float32)
               + bo[...])


def _tc_post(num, den, xb, rep, Wa, ba, Woa, Wox, bo, rb=1000):
    grid = (N // rb,)
    return pl.pallas_call(
        _post_body,
        grid=grid,
        in_specs=[
            pl.BlockSpec((rb, C), lambda i: (i, 0)),
            pl.BlockSpec((rb, H), lambda i: (i, 0)),
            pl.BlockSpec((rb, C), lambda i: (i, 0)),
            pl.BlockSpec((H, C), lambda i: (0, 0)),
            pl.BlockSpec((C, C), lambda i: (0, 0)),
            pl.BlockSpec((1, C), lambda i: (0, 0)),
            pl.BlockSpec((C, OUT), lambda i: (0, 0)),
            pl.BlockSpec((C, OUT), lambda i: (0, 0)),
            pl.BlockSpec((1, OUT), lambda i: (0, 0)),
        ],
        out_specs=pl.BlockSpec((rb, OUT), lambda i: (i, 0)),
        out_shape=jax.ShapeDtypeStruct((N, OUT), jnp.float32),
    )(num, den, xb, rep, Wa, ba, Woa, Wox, bo)


def kernel(x_book, x_film, x_music, ei_book_film, ei_film_music, ei_music_book,
           W_in_book, b_in_book, W_k_book, b_k_book, W_q_book, b_q_book,
           W_v_book, b_v_book, W_a_book, b_a_book, skip_book,
           W_in_film, b_in_film, W_k_film, b_k_film, W_q_film, b_q_film,
           W_v_film, b_v_film, W_a_film, b_a_film, skip_film,
           W_in_music, b_in_music, W_k_music, b_k_music, W_q_music, b_q_music,
           W_v_music, b_v_music, W_a_music, b_a_music, skip_music,
           a_rel_book_film, m_rel_book_film, p_rel_book_film,
           a_rel_film_music, m_rel_film_music, p_rel_film_music,
           a_rel_music_book, m_rel_music_book, p_rel_music_book,
           W_out, b_out):
    f32 = jnp.float32

    def fold(W, b, rel):
        We = jnp.einsum('chd,hdf->chf', W.reshape(C, H, D), rel).reshape(C, C)
        be = jnp.einsum('hd,hdf->hf', b.reshape(H, D), rel).reshape(C)
        return We, be

    Wk_eff, bk_eff = fold(W_k_music, b_k_music, a_rel_music_book)
    Wv_eff, bv_eff = fold(W_v_music, b_v_music, m_rel_music_book)
    scale = jnp.repeat(p_rel_music_book, D) / jnp.sqrt(f32(D))
    Wq_eff = W_q_book * scale[None, :]
    bq_eff = b_q_book * scale

    xb, qt, kvf = _tc_pre(
        x_book, x_music,
        W_in_book, b_in_book.reshape(1, C),
        W_in_music, b_in_music.reshape(1, C),
        Wq_eff, bq_eff.reshape(1, C),
        Wk_eff, bk_eff.reshape(1, C),
        Wv_eff, bv_eff.reshape(1, C))

    src = ei_music_book[0]
    dst = ei_music_book[1]
    npe = EPAD - E
    srcg = jnp.concatenate([src, jnp.zeros((npe,), jnp.int32)])
    dstg = jnp.concatenate([dst, jnp.zeros((npe,), jnp.int32)])
    dstp = jnp.concatenate([dst, jnp.full((npe,), 2 * NR - 1, jnp.int32)])
    dst0 = jnp.where(dstp < NR, dstp, SENT)
    dst1 = jnp.where(dstp >= NR, dstp - NR, SENT)
    dsts = jnp.concatenate([dst0, dst1])
    zrows = jnp.zeros((B, 2 * C), f32)

    kv_e, q_e = _sc_gather(kvf, qt, srcg, dstg)

    summ = (jnp.arange(C)[:, None] // D == jnp.arange(H)[None, :]).astype(f32)
    rep = summ.T
    rows = _tc_mid(kv_e, q_e, summ, rep)

    accn = _sc_scatter(rows, dsts, zrows, C, 0)
    accd = _sc_scatter(rows, dsts, zrows, H, C)

    num = accn[:N, :C]
    den = accd[:N, :H]

    beta = jax.nn.sigmoid(skip_book)
    Woa = W_out * beta
    Wox = W_out * (1.0 - beta)
    y = _tc_post(num, den, xb, rep, W_a_book, b_a_book.reshape(1, C),
                 Woa, Wox, b_out.reshape(1, OUT))
    return y


# speedup vs baseline: 58.7171x; 1.1344x over previous
import jax
import jax.numpy as jnp
from jax import lax
from jax.experimental import pallas as pl
from jax.experimental.pallas import tpu as pltpu
from jax.experimental.pallas import tpu_sc as plsc

N = 50000
E = 800000
C = 64
H = 8
D = 8
OUT = 8

B = 128
NSUB = 16
NW = 32
EPAD = 802816
CPW = EPAD // B // NW
CPT = EPAD // B // NSUB
ACCW = 72
NR = 25088
SENT = 2 ** 30
ZCH = NR // B
ZPT = -(-ZCH // NSUB)
RPT = NR // NSUB
EB = 2048
G = 4


def _pre_body(xb_ref, xm_ref, Wib, bib, Wim, bim, Wq, bq, Wk, bk, Wv, bv,
              xbo, qto, kvo):
    xb = jnp.maximum(jnp.dot(xb_ref[...], Wib[...],
                             preferred_element_type=jnp.float32) + bib[...], 0.0)
    xm = jnp.maximum(jnp.dot(xm_ref[...], Wim[...],
                             preferred_element_type=jnp.float32) + bim[...], 0.0)
    xbo[...] = xb
    qto[:, :C] = jnp.dot(xb, Wq[...], preferred_element_type=jnp.float32) + bq[...]
    qto[:, C:] = jnp.zeros_like(xb)
    kvo[:, :C] = jnp.dot(xm, Wk[...], preferred_element_type=jnp.float32) + bk[...]
    kvo[:, C:] = jnp.dot(xm, Wv[...], preferred_element_type=jnp.float32) + bv[...]


def _tc_pre(x_book, x_music, Wib, bib, Wim, bim, Wq, bq, Wk, bk, Wv, bv,
            rb=1000):
    grid = (N // rb,)
    row = pl.BlockSpec((rb, C), lambda i: (i, 0))
    mat = pl.BlockSpec((C, C), lambda i: (0, 0))
    vec = pl.BlockSpec((1, C), lambda i: (0, 0))
    return pl.pallas_call(
        _pre_body,
        grid=grid,
        in_specs=[row, row, mat, vec, mat, vec, mat, vec, mat, vec, mat, vec],
        out_specs=[row, pl.BlockSpec((rb, 2 * C), lambda i: (i, 0)),
                   pl.BlockSpec((rb, 2 * C), lambda i: (i, 0))],
        out_shape=[jax.ShapeDtypeStruct((N, C), jnp.float32),
                   jax.ShapeDtypeStruct((N, 2 * C), jnp.float32),
                   jax.ShapeDtypeStruct((N, 2 * C), jnp.float32)],
    )(x_book, x_music, Wib, bib, Wim, bim, Wq, bq, Wk, bk, Wv, bv)


def _sc_gather_body(kvf, qf, srcg, dstg, kv_e, q_e,
                    sidx, didx, kvb, qb, sem0, sem1):
    c = lax.axis_index("c")
    s = lax.axis_index("s")
    w = s * 2 + c

    def body(i, carry):
        base = (w * CPW + i * G) * B
        pltpu.sync_copy(srcg.at[pl.ds(base, G * B)], sidx)
        pltpu.sync_copy(dstg.at[pl.ds(base, G * B)], didx)
        for g in range(G):
            cp0 = pltpu.async_copy(kvf.at[sidx.at[pl.ds(g * B, B)]], kvb, sem0)
            cp1 = pltpu.async_copy(qf.at[didx.at[pl.ds(g * B, B)]], qb, sem1)
            cp0.wait()
            cp1.wait()
            pltpu.sync_copy(kvb, kv_e.at[pl.ds(base + g * B, B)])
            pltpu.sync_copy(qb, q_e.at[pl.ds(base + g * B, B)])
        return carry

    lax.fori_loop(0, CPW // G, body, 0)


def _sc_gather(kvf, qf, srcg, dstg):
    mesh = plsc.VectorSubcoreMesh(core_axis_name="c", subcore_axis_name="s")
    f = pl.kernel(
        _sc_gather_body,
        out_type=[jax.ShapeDtypeStruct((EPAD, 2 * C), jnp.float32),
                  jax.ShapeDtypeStruct((EPAD, 2 * C), jnp.float32)],
        mesh=mesh,
        scratch_types=[
            pltpu.VMEM((G * B,), jnp.int32),
            pltpu.VMEM((G * B,), jnp.int32),
            pltpu.VMEM((B, 2 * C), jnp.float32),
            pltpu.VMEM((B, 2 * C), jnp.float32),
            pltpu.SemaphoreType.DMA,
            pltpu.SemaphoreType.DMA,
        ],
    )
    return f(kvf, qf, srcg, dstg)


def _mid_body(kv_ref, q_ref, summ, rep, o_ref):
    kv = kv_ref[...]
    q = q_ref[...]
    t = q[:, :C] * kv[:, :C]
    w8 = jnp.exp(jnp.dot(t, summ[...], preferred_element_type=jnp.float32))
    wv = kv[:, C:] * jnp.dot(w8, rep[...], preferred_element_type=jnp.float32)
    o_ref[:, :C] = wv
    o_ref[:, C:C + H] = w8
    o_ref[:, C + H:] = jnp.zeros((wv.shape[0], C - H), jnp.float32)


def _tc_mid(kv_e, q_e, summ, rep):
    grid = (EPAD // EB,)
    return pl.pallas_call(
        _mid_body,
        grid=grid,
        in_specs=[
            pl.BlockSpec((EB, 2 * C), lambda i: (i, 0)),
            pl.BlockSpec((EB, 2 * C), lambda i: (i, 0)),
            pl.BlockSpec((C, H), lambda i: (0, 0)),
            pl.BlockSpec((H, C), lambda i: (0, 0)),
        ],
        out_specs=pl.BlockSpec((EB, 2 * C), lambda i: (i, 0)),
        out_shape=jax.ShapeDtypeStruct((EPAD, 2 * C), jnp.float32),
    )(kv_e, q_e, summ, rep)


def _make_scatter_body(width, coff, g):
    zch = NR // B
    zpt = -(-zch // NSUB)
    rpt = NR // NSUB

    def body(rows, dsts, zrows, acc_out, dsc, ob, zb, acc, sem0):
        c = lax.axis_index("c")
        s = lax.axis_index("s")

        pltpu.sync_copy(zrows.at[:, pl.ds(0, width)], zb)

        def zbody(i, carry):
            j = jnp.minimum(s + i * NSUB, zch - 1)
            pltpu.sync_copy(zb, acc.at[pl.ds(j * B, B)])
            return carry

        lax.fori_loop(0, zpt, zbody, 0)
        plsc.subcore_barrier()

        def ebody(i, carry):
            base = (s * CPT + i * g) * B
            pltpu.sync_copy(rows.at[pl.ds(base, g * B), pl.ds(coff, width)], ob)
            pltpu.sync_copy(dsts.at[pl.ds(c * EPAD + base, g * B)], dsc)
            for k in range(g):
                pltpu.sync_copy(
                    ob.at[pl.ds(k * B, B)],
                    acc.at[plsc.Indices(dsc.at[pl.ds(k * B, B)],
                                        ignored_value=SENT)],
                    add=True)
            return carry

        lax.fori_loop(0, CPT // g, ebody, 0)
        plsc.subcore_barrier()
        pltpu.sync_copy(acc.at[pl.ds(s * rpt, rpt)],
                        acc_out.at[pl.ds(c * NR + s * rpt, rpt), pl.ds(0, width)])

    return body


def _sc_scatter(rows, dsts, zrows, width, coff, g):
    mesh = plsc.VectorSubcoreMesh(core_axis_name="c", subcore_axis_name="s")
    f = pl.kernel(
        _make_scatter_body(width, coff, g),
        out_type=jax.ShapeDtypeStruct((2 * NR, 2 * C), jnp.float32),
        mesh=mesh,
        compiler_params=pltpu.CompilerParams(use_tc_tiling_on_sc=False),
        scratch_types=[
            pltpu.VMEM((g * B,), jnp.int32),
            pltpu.VMEM((g * B, width), jnp.float32),
            pltpu.VMEM((B, width), jnp.float32),
            pltpu.VMEM_SHARED((NR, width), jnp.float32),
            pltpu.SemaphoreType.DMA,
        ],
    )
    return f(rows, dsts, zrows)


def _post_body(num_ref, den_ref, xb_ref, rep, Wa, ba, Woa, Wox, bo, yo):
    den_exp = jnp.dot(den_ref[...], rep[...], preferred_element_type=jnp.float32)
    agg = num_ref[...] / (den_exp + 1e-16)
    o = jax.nn.gelu(jnp.dot(agg, Wa[...], preferred_element_type=jnp.float32)
                    + ba[...])
    yo[...] = (jnp.dot(o, Woa[...], preferred_element_type=jnp.float32)
               + jnp.dot(xb_ref[...], Wox[...], preferred_element_type=jnp.float32)
               + bo[...])


def _tc_post(num, den, xb, rep, Wa, ba, Woa, Wox, bo, rb=1000):
    grid = (N // rb,)
    return pl.pallas_call(
        _post_body,
        grid=grid,
        in_specs=[
            pl.BlockSpec((rb, C), lambda i: (i, 0)),
            pl.BlockSpec((rb, H), lambda i: (i, 0)),
            pl.BlockSpec((rb, C), lambda i: (i, 0)),
            pl.BlockSpec((H, C), lambda i: (0, 0)),
            pl.BlockSpec((C, C), lambda i: (0, 0)),
            pl.BlockSpec((1, C), lambda i: (0, 0)),
            pl.BlockSpec((C, OUT), lambda i: (0, 0)),
            pl.BlockSpec((C, OUT), lambda i: (0, 0)),
            pl.BlockSpec((1, OUT), lambda i: (0, 0)),
        ],
        out_specs=pl.BlockSpec((rb, OUT), lambda i: (i, 0)),
        out_shape=jax.ShapeDtypeStruct((N, OUT), jnp.float32),
    )(num, den, xb, rep, Wa, ba, Woa, Wox, bo)


def kernel(x_book, x_film, x_music, ei_book_film, ei_film_music, ei_music_book,
           W_in_book, b_in_book, W_k_book, b_k_book, W_q_book, b_q_book,
           W_v_book, b_v_book, W_a_book, b_a_book, skip_book,
           W_in_film, b_in_film, W_k_film, b_k_film, W_q_film, b_q_film,
           W_v_film, b_v_film, W_a_film, b_a_film, skip_film,
           W_in_music, b_in_music, W_k_music, b_k_music, W_q_music, b_q_music,
           W_v_music, b_v_music, W_a_music, b_a_music, skip_music,
           a_rel_book_film, m_rel_book_film, p_rel_book_film,
           a_rel_film_music, m_rel_film_music, p_rel_film_music,
           a_rel_music_book, m_rel_music_book, p_rel_music_book,
           W_out, b_out):
    f32 = jnp.float32

    def fold(W, b, rel):
        We = jnp.einsum('chd,hdf->chf', W.reshape(C, H, D), rel).reshape(C, C)
        be = jnp.einsum('hd,hdf->hf', b.reshape(H, D), rel).reshape(C)
        return We, be

    Wk_eff, bk_eff = fold(W_k_music, b_k_music, a_rel_music_book)
    Wv_eff, bv_eff = fold(W_v_music, b_v_music, m_rel_music_book)
    scale = jnp.repeat(p_rel_music_book, D) / jnp.sqrt(f32(D))
    Wq_eff = W_q_book * scale[None, :]
    bq_eff = b_q_book * scale

    xb, qt, kvf = _tc_pre(
        x_book, x_music,
        W_in_book, b_in_book.reshape(1, C),
        W_in_music, b_in_music.reshape(1, C),
        Wq_eff, bq_eff.reshape(1, C),
        Wk_eff, bk_eff.reshape(1, C),
        Wv_eff, bv_eff.reshape(1, C))

    src = ei_music_book[0]
    dst = ei_music_book[1]
    npe = EPAD - E
    srcg = jnp.concatenate([src, jnp.zeros((npe,), jnp.int32)])
    dstg = jnp.concatenate([dst, jnp.zeros((npe,), jnp.int32)])
    dstp = jnp.concatenate([dst, jnp.full((npe,), 2 * NR - 1, jnp.int32)])
    dst0 = jnp.where(dstp < NR, dstp, SENT)
    dst1 = jnp.where(dstp >= NR, dstp - NR, SENT)
    dsts = jnp.concatenate([dst0, dst1])
    zrows = jnp.zeros((B, 2 * C), f32)

    kv_e, q_e = _sc_gather(kvf, qt, srcg, dstg)

    summ = (jnp.arange(C)[:, None] // D == jnp.arange(H)[None, :]).astype(f32)
    rep = summ.T
    rows = _tc_mid(kv_e, q_e, summ, rep)

    accn = _sc_scatter(rows, dsts, zrows, C, 0, 1)
    accd = _sc_scatter(rows, dsts, zrows, H, C, 4)

    num = accn[:N, :C]
    den = accd[:N, :H]

    beta = jax.nn.sigmoid(skip_book)
    Woa = W_out * beta
    Wox = W_out * (1.0 - beta)
    y = _tc_post(num, den, xb, rep, W_a_book, b_a_book.reshape(1, C),
                 Woa, Wox, b_out.reshape(1, OUT))
    return y


# speedup vs baseline: 58.7285x; 1.0002x over previous
import jax
import jax.numpy as jnp
from jax import lax
from jax.experimental import pallas as pl
from jax.experimental.pallas import tpu as pltpu
from jax.experimental.pallas import tpu_sc as plsc

N = 50000
E = 800000
C = 64
H = 8
D = 8
OUT = 8

B = 128
NSUB = 16
NW = 32
EPAD = 802816
CPW = EPAD // B // NW
CPT = EPAD // B // NSUB
NR = 25088
SENT = 2 ** 30
EB = 2048
G = 4


def _pre_body(xb_ref, xm_ref, Wib, bib, Wim, bim, Wq, bq, Wk, bk, Wv, bv,
              xbo, qto, kvo):
    xb = jnp.maximum(jnp.dot(xb_ref[...], Wib[...],
                             preferred_element_type=jnp.float32) + bib[...], 0.0)
    xm = jnp.maximum(jnp.dot(xm_ref[...], Wim[...],
                             preferred_element_type=jnp.float32) + bim[...], 0.0)
    xbo[...] = xb
    qto[:, :C] = jnp.dot(xb, Wq[...], preferred_element_type=jnp.float32) + bq[...]
    qto[:, C:] = jnp.zeros_like(xb)
    kvo[:, :C] = jnp.dot(xm, Wk[...], preferred_element_type=jnp.float32) + bk[...]
    kvo[:, C:] = jnp.dot(xm, Wv[...], preferred_element_type=jnp.float32) + bv[...]


def _tc_pre(x_book, x_music, Wib, bib, Wim, bim, Wq, bq, Wk, bk, Wv, bv,
            rb=1000):
    grid = (N // rb,)
    row = pl.BlockSpec((rb, C), lambda i: (i, 0))
    mat = pl.BlockSpec((C, C), lambda i: (0, 0))
    vec = pl.BlockSpec((1, C), lambda i: (0, 0))
    return pl.pallas_call(
        _pre_body,
        grid=grid,
        in_specs=[row, row, mat, vec, mat, vec, mat, vec, mat, vec, mat, vec],
        out_specs=[row, pl.BlockSpec((rb, 2 * C), lambda i: (i, 0)),
                   pl.BlockSpec((rb, 2 * C), lambda i: (i, 0))],
        out_shape=[jax.ShapeDtypeStruct((N, C), jnp.float32),
                   jax.ShapeDtypeStruct((N, 2 * C), jnp.float32),
                   jax.ShapeDtypeStruct((N, 2 * C), jnp.float32)],
    )(x_book, x_music, Wib, bib, Wim, bim, Wq, bq, Wk, bk, Wv, bv)


def _sc_gather_body(kvf, qf, srcg, dstg, kv_e, q_e,
                    sidx, didx, kvb, qb, sem0, sem1):
    c = lax.axis_index("c")
    s = lax.axis_index("s")
    w = s * 2 + c

    def body(i, carry):
        base = (w * CPW + i * G) * B
        pltpu.sync_copy(srcg.at[pl.ds(base, G * B)], sidx)
        pltpu.sync_copy(dstg.at[pl.ds(base, G * B)], didx)
        for g in range(G):
            cp0 = pltpu.async_copy(kvf.at[sidx.at[pl.ds(g * B, B)]], kvb, sem0)
            cp1 = pltpu.async_copy(qf.at[didx.at[pl.ds(g * B, B)]], qb, sem1)
            cp0.wait()
            cp1.wait()
            pltpu.sync_copy(kvb, kv_e.at[pl.ds(base + g * B, B)])
            pltpu.sync_copy(qb, q_e.at[pl.ds(base + g * B, B)])
        return carry

    lax.fori_loop(0, CPW // G, body, 0)


def _sc_gather(kvf, qf, srcg, dstg):
    mesh = plsc.VectorSubcoreMesh(core_axis_name="c", subcore_axis_name="s")
    f = pl.kernel(
        _sc_gather_body,
        out_type=[jax.ShapeDtypeStruct((EPAD, 2 * C), jnp.float32),
                  jax.ShapeDtypeStruct((EPAD, 2 * C), jnp.float32)],
        mesh=mesh,
        scratch_types=[
            pltpu.VMEM((G * B,), jnp.int32),
            pltpu.VMEM((G * B,), jnp.int32),
            pltpu.VMEM((B, 2 * C), jnp.float32),
            pltpu.VMEM((B, 2 * C), jnp.float32),
            pltpu.SemaphoreType.DMA,
            pltpu.SemaphoreType.DMA,
        ],
    )
    return f(kvf, qf, srcg, dstg)


def _mid_body(kv_ref, q_ref, summ, rep, o_ref):
    kv = kv_ref[...]
    q = q_ref[...]
    t = q[:, :C] * kv[:, :C]
    w8 = jnp.exp(jnp.dot(t, summ[...], preferred_element_type=jnp.float32))
    wv = kv[:, C:] * jnp.dot(w8, rep[...], preferred_element_type=jnp.float32)
    o_ref[:, :C] = wv
    o_ref[:, C:C + H] = w8
    o_ref[:, C + H:] = jnp.zeros((wv.shape[0], C - H), jnp.float32)


def _tc_mid(kv_e, q_e, summ, rep):
    grid = (EPAD // EB,)
    return pl.pallas_call(
        _mid_body,
        grid=grid,
        in_specs=[
            pl.BlockSpec((EB, 2 * C), lambda i: (i, 0)),
            pl.BlockSpec((EB, 2 * C), lambda i: (i, 0)),
            pl.BlockSpec((C, H), lambda i: (0, 0)),
            pl.BlockSpec((H, C), lambda i: (0, 0)),
        ],
        out_specs=pl.BlockSpec((EB, 2 * C), lambda i: (i, 0)),
        out_shape=jax.ShapeDtypeStruct((EPAD, 2 * C), jnp.float32),
    )(kv_e, q_e, summ, rep)


def _make_scatter_body(width, coff, g):
    zch = NR // B
    zpt = -(-zch // NSUB)
    rpt = NR // NSUB

    def body(rows, dsts, zrows, acc_out, dsc, ob, zb, acc, sem0):
        c = lax.axis_index("c")
        s = lax.axis_index("s")

        pltpu.sync_copy(zrows.at[:, pl.ds(0, width)], zb)

        def zbody(i, carry):
            j = jnp.minimum(s + i * NSUB, zch - 1)
            pltpu.sync_copy(zb, acc.at[pl.ds(j * B, B)])
            return carry

        lax.fori_loop(0, zpt, zbody, 0)
        plsc.subcore_barrier()

        def ebody(i, carry):
            base = (s * CPT + i * g) * B
            pltpu.sync_copy(rows.at[pl.ds(base, g * B), pl.ds(coff, width)], ob)
            pltpu.sync_copy(dsts.at[pl.ds(c * EPAD + base, g * B)], dsc)
            for k in range(g):
                pltpu.sync_copy(
                    ob.at[pl.ds(k * B, B)],
                    acc.at[plsc.Indices(dsc.at[pl.ds(k * B, B)],
                                        ignored_value=SENT)],
                    add=True)
            return carry

        lax.fori_loop(0, CPT // g, ebody, 0)
        plsc.subcore_barrier()
        pltpu.sync_copy(acc.at[pl.ds(s * rpt, rpt)],
                        acc_out.at[pl.ds(c * NR + s * rpt, rpt), pl.ds(0, width)])

    return body


def _sc_scatter(rows, dsts, zrows, width, coff, g):
    mesh = plsc.VectorSubcoreMesh(core_axis_name="c", subcore_axis_name="s")
    f = pl.kernel(
        _make_scatter_body(width, coff, g),
        out_type=jax.ShapeDtypeStruct((2 * NR, 2 * C), jnp.float32),
        mesh=mesh,
        compiler_params=pltpu.CompilerParams(use_tc_tiling_on_sc=False),
        scratch_types=[
            pltpu.VMEM((g * B,), jnp.int32),
            pltpu.VMEM((g * B, width), jnp.float32),
            pltpu.VMEM((B, width), jnp.float32),
            pltpu.VMEM_SHARED((NR, width), jnp.float32),
            pltpu.SemaphoreType.DMA,
        ],
    )
    return f(rows, dsts, zrows)


def _post_body(num_ref, den_ref, xb_ref, rep, Wa, ba, Woa, Wox, bo, yo):
    den_exp = jnp.dot(den_ref[...], rep[...], preferred_element_type=jnp.float32)
    agg = num_ref[...] / (den_exp + 1e-16)
    o = jax.nn.gelu(jnp.dot(agg, Wa[...], preferred_element_type=jnp.float32)
                    + ba[...])
    yo[...] = (jnp.dot(o, Woa[...], preferred_element_type=jnp.float32)
               + jnp.dot(xb_ref[...], Wox[...], preferred_element_type=jnp.float32)
               + bo[...])


def _tc_post(num, den, xb, rep, Wa, ba, Woa, Wox, bo, rb=1000):
    grid = (N // rb,)
    return pl.pallas_call(
        _post_body,
        grid=grid,
        in_specs=[
            pl.BlockSpec((rb, C), lambda i: (i, 0)),
            pl.BlockSpec((rb, H), lambda i: (i, 0)),
            pl.BlockSpec((rb, C), lambda i: (i, 0)),
            pl.BlockSpec((H, C), lambda i: (0, 0)),
            pl.BlockSpec((C, C), lambda i: (0, 0)),
            pl.BlockSpec((1, C), lambda i: (0, 0)),
            pl.BlockSpec((C, OUT), lambda i: (0, 0)),
            pl.BlockSpec((C, OUT), lambda i: (0, 0)),
            pl.BlockSpec((1, OUT), lambda i: (0, 0)),
        ],
        out_specs=pl.BlockSpec((rb, OUT), lambda i: (i, 0)),
        out_shape=jax.ShapeDtypeStruct((N, OUT), jnp.float32),
    )(num, den, xb, rep, Wa, ba, Woa, Wox, bo)


def kernel(x_book, x_film, x_music, ei_book_film, ei_film_music, ei_music_book,
           W_in_book, b_in_book, W_k_book, b_k_book, W_q_book, b_q_book,
           W_v_book, b_v_book, W_a_book, b_a_book, skip_book,
           W_in_film, b_in_film, W_k_film, b_k_film, W_q_film, b_q_film,
           W_v_film, b_v_film, W_a_film, b_a_film, skip_film,
           W_in_music, b_in_music, W_k_music, b_k_music, W_q_music, b_q_music,
           W_v_music, b_v_music, W_a_music, b_a_music, skip_music,
           a_rel_book_film, m_rel_book_film, p_rel_book_film,
           a_rel_film_music, m_rel_film_music, p_rel_film_music,
           a_rel_music_book, m_rel_music_book, p_rel_music_book,
           W_out, b_out):
    f32 = jnp.float32

    def fold(W, b, rel):
        We = jnp.einsum('chd,hdf->chf', W.reshape(C, H, D), rel).reshape(C, C)
        be = jnp.einsum('hd,hdf->hf', b.reshape(H, D), rel).reshape(C)
        return We, be

    Wk_eff, bk_eff = fold(W_k_music, b_k_music, a_rel_music_book)
    Wv_eff, bv_eff = fold(W_v_music, b_v_music, m_rel_music_book)
    scale = jnp.repeat(p_rel_music_book, D) / jnp.sqrt(f32(D))
    Wq_eff = W_q_book * scale[None, :]
    bq_eff = b_q_book * scale

    xb, qt, kvf = _tc_pre(
        x_book, x_music,
        W_in_book, b_in_book.reshape(1, C),
        W_in_music, b_in_music.reshape(1, C),
        Wq_eff, bq_eff.reshape(1, C),
        Wk_eff, bk_eff.reshape(1, C),
        Wv_eff, bv_eff.reshape(1, C))

    src = ei_music_book[0]
    dst = ei_music_book[1]
    npe = EPAD - E
    srcg = jnp.concatenate([src, jnp.zeros((npe,), jnp.int32)])
    dstg = jnp.concatenate([dst, jnp.zeros((npe,), jnp.int32)])
    dstp = jnp.concatenate([dst, jnp.full((npe,), 2 * NR - 1, jnp.int32)])
    dst0 = jnp.where(dstp < NR, dstp, SENT)
    dst1 = jnp.where(dstp >= NR, dstp - NR, SENT)
    dsts = jnp.concatenate([dst0, dst1])
    zrows = jnp.zeros((B, 2 * C), f32)

    kv_e, q_e = _sc_gather(kvf, qt, srcg, dstg)

    summ = (jnp.arange(C)[:, None] // D == jnp.arange(H)[None, :]).astype(f32)
    rep = summ.T
    rows = _tc_mid(kv_e, q_e, summ, rep)

    accn = _sc_scatter(rows, dsts, zrows, C, 0, 1)
    accd = _sc_scatter(rows, dsts, zrows, H, C, 4)

    num = accn[:N, :C]
    den = accd[:N, :H]

    beta = jax.nn.sigmoid(skip_book)
    Woa = W_out * beta
    Wox = W_out * (1.0 - beta)
    y = _tc_post(num, den, xb, rep, W_a_book, b_a_book.reshape(1, C),
                 Woa, Wox, b_out.reshape(1, OUT))
    return y
